# baseline (speedup 1.0000x reference)
import functools

import jax
import jax.numpy as jnp
import numpy as np
from jax import lax
from jax.experimental import pallas as pl
from jax.experimental.pallas import tpu as pltpu
from jax.experimental.pallas import tpu_sc as plsc

LOG2 = float(np.log(2.0))
NN = 50000
NE = 800000
T_E = 4000
T_N = 2000


def _sp2(x):
    return jnp.logaddexp(x, 0.0) - LOG2


def _bf16x3(a, b):
    ah = a.astype(jnp.bfloat16)
    al = (a - ah.astype(jnp.float32)).astype(jnp.bfloat16)
    bh = b.astype(jnp.bfloat16)
    bl = (b - bh.astype(jnp.float32)).astype(jnp.bfloat16)
    d = lambda x, y: lax.dot_general(x, y, (((1,), (0,)), ((), ())),
                                     preferred_element_type=jnp.float32)
    return d(ah, bh) + d(ah, bl) + d(al, bh)


def _dot(a, b):
    return lax.dot_general(a.astype(jnp.bfloat16), b.astype(jnp.bfloat16),
                           (((1,), (0,)), ((), ())),
                           preferred_element_type=jnp.float32)


def _mlp2_body(x_ref, w1, b1, w2, b2, o_ref):
    h = _sp2(_dot(x_ref[...], w1[...]) + b1[...])
    o_ref[...] = _sp2(_dot(h, w2[...]) + b2[...])


def _mlp2(x, w1, b1, w2, b2, tile):
    n, d = x.shape
    dh, do = w1.shape[1], w2.shape[1]
    return pl.pallas_call(
        _mlp2_body,
        grid=(n // tile,),
        in_specs=[
            pl.BlockSpec((tile, d), lambda i: (i, 0)),
            pl.BlockSpec((d, dh), lambda i: (0, 0)),
            pl.BlockSpec((1, dh), lambda i: (0, 0)),
            pl.BlockSpec((dh, do), lambda i: (0, 0)),
            pl.BlockSpec((1, do), lambda i: (0, 0)),
        ],
        out_specs=pl.BlockSpec((tile, do), lambda i: (i, 0)),
        out_shape=jax.ShapeDtypeStruct((n, do), jnp.float32),
    )(x, w1, b1, w2, b2)


def _node_enc_body(nf_ref, emb, w1, b1, w2, b2, o_ref):
    nf = nf_ref[0, 0, :]
    oh = (lax.broadcasted_iota(jnp.int32, (T_N, 96), 1) == nf[:, None])
    h0 = _dot(oh.astype(jnp.float32), emb[...])
    h = _sp2(_dot(h0, w1[...]) + b1[...])
    o_ref[...] = _sp2(_dot(h, w2[...]) + b2[...])


def _node_enc(node_feat, emb_pad, w1, b1, w2, b2):
    nf3 = node_feat.reshape(NN // T_N, 1, T_N)
    return pl.pallas_call(
        _node_enc_body,
        grid=(NN // T_N,),
        in_specs=[
            pl.BlockSpec((1, 1, T_N), lambda i: (i, 0, 0)),
            pl.BlockSpec((96, 16), lambda i: (0, 0)),
            pl.BlockSpec((16, 64), lambda i: (0, 0)),
            pl.BlockSpec((1, 64), lambda i: (0, 0)),
            pl.BlockSpec((64, 32), lambda i: (0, 0)),
            pl.BlockSpec((1, 32), lambda i: (0, 0)),
        ],
        out_specs=pl.BlockSpec((T_N, 32), lambda i: (i, 0)),
        out_shape=jax.ShapeDtypeStruct((NN, 32), jnp.float32),
    )(nf3, emb_pad, w1, b1, w2, b2)


def _econv_body(dense, vs_ref, vd_ref, e0_ref, u_ref,
                wa, wb, wc, wd, b1, w2, b2, w3, b3,
                wd1, bd1, wd2, bd2, enew_ref, eres_ref, acc_ref):
    i = pl.program_id(0)
    if dense:
        ed = _sp2(_dot(e0_ref[...], wd1[...]) + bd1[...])
        ed = _sp2(_dot(ed, wd2[...]) + bd2[...])
    else:
        ed = e0_ref[...]
    h = (_dot(vs_ref[...], wa[...]) + _dot(vd_ref[...], wb[...])
         + _dot(ed, wc[...]) + _dot(u_ref[...], wd[...]) + b1[...])
    h = _sp2(h)
    h = _sp2(_dot(h, w2[...]) + b2[...])
    en = _sp2(_dot(h, w3[...]) + b3[...])
    enew_ref[...] = en
    eres_ref[...] = en + e0_ref[...]
    part = jnp.sum(en, axis=0, keepdims=True)

    @pl.when(i == 0)
    def _():
        acc_ref[...] = part

    @pl.when(i > 0)
    def _():
        acc_ref[...] += part


def _edge_conv(vpair, e0, u, p, dense_p):
    (w1, b1), (w2, b2), (w3, b3) = p
    wa, wb, wc, wd = w1[0:32], w1[32:64], w1[64:96], w1[96:128]
    if dense_p:
        (wd1, bd1), (wd2, bd2) = dense_p
    else:
        wd1 = jnp.zeros((32, 64), jnp.float32)
        bd1 = jnp.zeros((64,), jnp.float32)
        wd2 = jnp.zeros((64, 32), jnp.float32)
        bd2 = jnp.zeros((32,), jnp.float32)
    full = lambda s: pl.BlockSpec(s, lambda i: (0, 0))
    return pl.pallas_call(
        functools.partial(_econv_body, bool(dense_p)),
        grid=(NE // T_E,),
        in_specs=[
            pl.BlockSpec((T_E, 32), lambda i: (i, 0)),
            pl.BlockSpec((T_E, 32), lambda i: (i + NE // T_E, 0)),
            pl.BlockSpec((T_E, 32), lambda i: (i, 0)),
            full((1, 32)),
            full((32, 64)), full((32, 64)), full((32, 64)), full((32, 64)),
            full((1, 64)), full((64, 64)), full((1, 64)),
            full((64, 32)), full((1, 32)),
            full((32, 64)), full((1, 64)), full((64, 32)), full((1, 32)),
        ],
        out_specs=[
            pl.BlockSpec((T_E, 32), lambda i: (i, 0)),
            pl.BlockSpec((T_E, 32), lambda i: (i, 0)),
            pl.BlockSpec((1, 32), lambda i: (0, 0)),
        ],
        out_shape=[
            jax.ShapeDtypeStruct((NE, 32), jnp.float32),
            jax.ShapeDtypeStruct((NE, 32), jnp.float32),
            jax.ShapeDtypeStruct((1, 32), jnp.float32),
        ],
    )(vpair, vpair, e0, u.reshape(1, 32),
      wa, wb, wc, wd, b1.reshape(1, 64), w2, b2.reshape(1, 64),
      w3, b3.reshape(1, 32),
      wd1, bd1.reshape(1, 64), wd2, bd2.reshape(1, 32))


def _nconv_body(v_ref, v0_ref, es0_ref, es1_ref, c0_ref, c1_ref, u_ref,
                wa, wb, wc, b1, w2, b2, w3, b3, vres_ref, acc_ref):
    i = pl.program_id(0)
    cnt = jnp.maximum(c0_ref[0] + c1_ref[0], 1.0)[:, 0:1]
    ve = (es0_ref[0] + es1_ref[0]) / cnt
    h = (_dot(v_ref[...], wa[...]) + _dot(ve, wb[...])
         + _dot(u_ref[...], wc[...]) + b1[...])
    h = _sp2(h)
    h = _sp2(_dot(h, w2[...]) + b2[...])
    vn = _sp2(_dot(h, w3[...]) + b3[...])
    vres_ref[...] = vn + v0_ref[...]
    part = jnp.sum(vn, axis=0, keepdims=True)

    @pl.when(i == 0)
    def _():
        acc_ref[...] = part

    @pl.when(i > 0)
    def _():
        acc_ref[...] += part


def _node_conv(v_in, v0, esum, cnts, u, p):
    (w1, b1), (w2, b2), (w3, b3) = p
    wa, wb, wc = w1[0:32], w1[32:64], w1[64:96]
    full = lambda s: pl.BlockSpec(s, lambda i: (0, 0))
    return pl.pallas_call(
        _nconv_body,
        grid=(NN // T_N,),
        in_specs=[
            pl.BlockSpec((T_N, 32), lambda i: (i, 0)),
            pl.BlockSpec((T_N, 32), lambda i: (i, 0)),
            pl.BlockSpec((1, T_N, 32), lambda i: (0, i, 0)),
            pl.BlockSpec((1, T_N, 32), lambda i: (1, i, 0)),
            pl.BlockSpec((1, T_N, 16), lambda i: (0, i, 0)),
            pl.BlockSpec((1, T_N, 16), lambda i: (1, i, 0)),
            full((1, 32)),
            full((32, 64)), full((32, 64)), full((32, 64)),
            full((1, 64)), full((64, 64)), full((1, 64)),
            full((64, 32)), full((1, 32)),
        ],
        out_specs=[
            pl.BlockSpec((T_N, 32), lambda i: (i, 0)),
            pl.BlockSpec((1, 32), lambda i: (0, 0)),
        ],
        out_shape=[
            jax.ShapeDtypeStruct((NN, 32), jnp.float32),
            jax.ShapeDtypeStruct((1, 32), jnp.float32),
        ],
    )(v_in, v0, esum, esum, cnts, cnts, u.reshape(1, 32),
      wa, wb, wc, b1.reshape(1, 64), w2, b2.reshape(1, 64),
      w3, b3.reshape(1, 32))


def _s2s_body(f_ref, q_ref, o_ref, m_sc, z_sc, r_sc):
    i = pl.program_id(0)

    @pl.when(i == 0)
    def _():
        m_sc[...] = jnp.full((1, 32), -3.0e38, jnp.float32)
        z_sc[...] = jnp.zeros((1, 32), jnp.float32)
        r_sc[...] = jnp.zeros((1, 32), jnp.float32)

    f = f_ref[...]
    s = _dot(f, q_ref[...].reshape(32, 1))[:, 0]
    m_old = jnp.max(m_sc[...])
    m_new = jnp.maximum(m_old, jnp.max(s))
    scale = jnp.exp(jnp.full((1, 32), m_old - m_new, jnp.float32))
    es = jnp.exp(s - m_new)
    r_sc[...] = r_sc[...] * scale + jnp.sum(f * es[:, None], axis=0,
                                            keepdims=True)
    z_sc[...] = z_sc[...] * scale + jnp.sum(es)
    m_sc[...] = jnp.full((1, 32), m_new, jnp.float32)
    o_ref[...] = r_sc[...] / z_sc[...]


def _s2s_pass(feat, q, tile):
    n = feat.shape[0]
    return pl.pallas_call(
        _s2s_body,
        grid=(n // tile,),
        in_specs=[
            pl.BlockSpec((tile, 32), lambda i: (i, 0)),
            pl.BlockSpec((1, 32), lambda i: (0, 0)),
        ],
        out_specs=pl.BlockSpec((1, 32), lambda i: (0, 0)),
        out_shape=jax.ShapeDtypeStruct((1, 32), jnp.float32),
        scratch_shapes=[pltpu.VMEM((1, 32), jnp.float32)] * 3,
    )(feat, q)


def _lstm_cell(x, h, c, p):
    wih, whh, bih, bhh = p
    g = x @ wih.T + bih + h @ whh.T + bhh
    i, f, gg, o = jnp.split(g, 4, axis=-1)
    c = jax.nn.sigmoid(f) * c + jax.nn.sigmoid(i) * jnp.tanh(gg)
    h = jax.nn.sigmoid(o) * jnp.tanh(c)
    return h, c


def _set2set(feat, p, tile):
    h = jnp.zeros((1, 32), jnp.float32)
    c = jnp.zeros((1, 32), jnp.float32)
    q_star = jnp.zeros((1, 64), jnp.float32)
    for _ in range(2):
        h, c = _lstm_cell(q_star, h, c, p)
        readout = _s2s_pass(feat, h, tile)
        q_star = jnp.concatenate([h, readout], axis=-1)
    return q_star


_MESH = dict(core_axis_name="c", subcore_axis_name="s")
_SC_PARAMS = pltpu.CompilerParams(use_tc_tiling_on_sc=False)


def _sc_gather(table, idx):
    ni = idx.shape[0]
    nch = ni // 128
    tmax = (nch + 31) // 32
    mesh = plsc.VectorSubcoreMesh(**_MESH)

    @functools.partial(
        pl.kernel, mesh=mesh, compiler_params=_SC_PARAMS,
        out_type=jax.ShapeDtypeStruct((ni, 32), jnp.float32),
        scratch_types=[
            pltpu.VMEM((128,), jnp.int32),
            pltpu.VMEM((128, 32), jnp.float32),
            pltpu.SemaphoreType.DMA,
        ],
    )
    def k(table_hbm, idx_hbm, out_hbm, idx_v, rows_v, sem):
        w = lax.axis_index("s") * 2 + lax.axis_index("c")

        @pl.loop(0, tmax)
        def _(t):
            ch = w + t * 32

            @pl.when(ch < nch)
            def _():
                base = ch * 128
                pltpu.sync_copy(idx_hbm.at[pl.ds(base, 128)], idx_v)
                pltpu.async_copy(table_hbm.at[idx_v], rows_v, sem).wait()
                pltpu.sync_copy(rows_v, out_hbm.at[pl.ds(base, 128)])

    return k(table, idx)


def _sc_scatter(values, idx, zeros400):
    nch = NE // 128
    tmax = (nch + 31) // 32
    mesh = plsc.VectorSubcoreMesh(**_MESH)

    @functools.partial(
        pl.kernel, mesh=mesh, compiler_params=_SC_PARAMS,
        out_type=jax.ShapeDtypeStruct((2, NN, 32), jnp.float32),
        scratch_types=[
            pltpu.VMEM((128,), jnp.int32),
            pltpu.VMEM((128, 32), jnp.float32),
            pltpu.VMEM_SHARED((NN, 32), jnp.float32),
            pltpu.SemaphoreType.DMA,
        ],
    )
    def k(val_hbm, idx_hbm, z_hbm, out_hbm, idx_v, rows_v, acc_sh, sem):
        cid = lax.axis_index("c")
        sid = lax.axis_index("s")
        w = sid * 2 + cid

        @pl.loop(0, 8)
        def _(t):
            cz = sid + t * 16

            @pl.when(cz < 125)
            def _():
                pltpu.sync_copy(z_hbm, acc_sh.at[pl.ds(cz * 400, 400)])

        plsc.subcore_barrier()

        @pl.loop(0, tmax)
        def _(t):
            ch = w + t * 32

            @pl.when(ch < nch)
            def _():
                base = ch * 128
                pltpu.sync_copy(idx_hbm.at[pl.ds(base, 128)], idx_v)
                pltpu.sync_copy(val_hbm.at[pl.ds(base, 128)], rows_v)
                pltpu.sync_copy(rows_v, acc_sh.at[idx_v], add=True)

        plsc.subcore_barrier()

        @pl.loop(0, 8)
        def _(t):
            co = sid + t * 16

            @pl.when(co < 125)
            def _():
                pltpu.sync_copy(acc_sh.at[pl.ds(co * 400, 400)],
                                out_hbm.at[cid, pl.ds(co * 400, 400)])

    return k(values, idx, zeros400)


def _sc_count(idx, ones128, zeros400):
    nch = NE // 128
    tmax = (nch + 31) // 32
    mesh = plsc.VectorSubcoreMesh(**_MESH)

    @functools.partial(
        pl.kernel, mesh=mesh, compiler_params=_SC_PARAMS,
        out_type=jax.ShapeDtypeStruct((2, NN, 16), jnp.float32),
        scratch_types=[
            pltpu.VMEM((128,), jnp.int32),
            pltpu.VMEM((128, 16), jnp.float32),
            pltpu.VMEM_SHARED((NN, 16), jnp.float32),
            pltpu.SemaphoreType.DMA,
        ],
    )
    def k(idx_hbm, ones_hbm, z_hbm, out_hbm, idx_v, ones_v, acc_sh, sem):
        cid = lax.axis_index("c")
        sid = lax.axis_index("s")
        w = sid * 2 + cid
        pltpu.sync_copy(ones_hbm, ones_v)

        @pl.loop(0, 8)
        def _(t):
            cz = sid + t * 16

            @pl.when(cz < 125)
            def _():
                pltpu.sync_copy(z_hbm, acc_sh.at[pl.ds(cz * 400, 400)])

        plsc.subcore_barrier()

        @pl.loop(0, tmax)
        def _(t):
            ch = w + t * 32

            @pl.when(ch < nch)
            def _():
                base = ch * 128
                pltpu.sync_copy(idx_hbm.at[pl.ds(base, 128)], idx_v)
                pltpu.sync_copy(ones_v, acc_sh.at[idx_v], add=True)

        plsc.subcore_barrier()

        @pl.loop(0, 8)
        def _(t):
            co = sid + t * 16

            @pl.when(co < 125)
            def _():
                pltpu.sync_copy(acc_sh.at[pl.ds(co * 400, 400)],
                                out_hbm.at[cid, pl.ds(co * 400, 400)])

    return k(idx, ones128, zeros400)


def _mlp_jnp(x, layers, activate_last=True):
    n = len(layers)
    for i, (w, b) in enumerate(layers):
        x = x @ w + b
        if i < n - 1 or activate_last:
            x = _sp2(x)
    return x


def kernel(edge_index, node_feat, edge_feat, state_feat, params):
    p = params
    idx_all = edge_index.reshape(2 * NE).astype(jnp.int32)
    dst = edge_index[1].astype(jnp.int32)

    zeros400_32 = jnp.zeros((400, 32), jnp.float32)
    zeros400_16 = jnp.zeros((400, 16), jnp.float32)
    ones128_16 = jnp.ones((128, 16), jnp.float32)

    (we1, be1), (we2, be2) = p['edge_enc']
    e = _mlp2(edge_feat, we1, be1.reshape(1, -1), we2, be2.reshape(1, -1), T_E)
    emb_pad = jnp.zeros((96, 16), jnp.float32).at[0:89].set(p['node_emb'])
    (wn1, bn1), (wn2, bn2) = p['node_enc']
    v = _node_enc(node_feat.astype(jnp.int32), emb_pad,
                  wn1, bn1.reshape(1, -1), wn2, bn2.reshape(1, -1))
    u = _mlp_jnp(state_feat, p['state_enc'])

    cnts = _sc_count(dst, ones128_16, zeros400_16)

    for blk in p['blocks']:
        e0, v0, u0 = e, v, u
        if blk['dense_n']:
            (wv1, bv1), (wv2, bv2) = blk['dense_n']
            v_d = _mlp2(v, wv1, bv1.reshape(1, -1), wv2, bv2.reshape(1, -1),
                        T_N)
            u_d = _mlp_jnp(u, blk['dense_s'])
        else:
            v_d, u_d = v, u

        vpair = _sc_gather(v_d, idx_all)
        e_new, e_res, e_acc = _edge_conv(vpair, e0, u_d, blk['conv_e'],
                                         blk['dense_e'])
        esum = _sc_scatter(e_new, dst, zeros400_32)
        v_new, v_acc = _node_conv(v_d, v0, esum, cnts, u_d, blk['conv_n'])

        e_mean = e_acc / float(NE)
        v_mean = v_acc / float(NN)
        u_new = _mlp_jnp(jnp.concatenate([u_d, e_mean, v_mean], axis=-1),
                         blk['conv_s'])
        e, v, u = e_res, v_new, u_new + u0

    node_vec = _set2set(v, p['lstm_node'], T_N)
    edge_vec = _set2set(e, p['lstm_edge'], T_E)
    vec = jnp.concatenate([node_vec[0], edge_vec[0], u[0]], axis=-1)
    out = _mlp_jnp(vec[None, :], p['out_proj'], activate_last=False)
    return out[0]


# speedup vs baseline: 1.0672x; 1.0672x over previous
import functools

import jax
import jax.numpy as jnp
import numpy as np
from jax import lax
from jax.experimental import pallas as pl
from jax.experimental.pallas import tpu as pltpu
from jax.experimental.pallas import tpu_sc as plsc

LOG2 = float(np.log(2.0))
NN = 50000
NE = 800000
T_E = 4000
T_N = 2000


def _sp2(x):
    return jnp.logaddexp(x, 0.0) - LOG2


def _bf16x3(a, b):
    ah = a.astype(jnp.bfloat16)
    al = (a - ah.astype(jnp.float32)).astype(jnp.bfloat16)
    bh = b.astype(jnp.bfloat16)
    bl = (b - bh.astype(jnp.float32)).astype(jnp.bfloat16)
    d = lambda x, y: lax.dot_general(x, y, (((1,), (0,)), ((), ())),
                                     preferred_element_type=jnp.float32)
    return d(ah, bh) + d(ah, bl) + d(al, bh)


def _dot(a, b):
    return lax.dot_general(a.astype(jnp.bfloat16), b.astype(jnp.bfloat16),
                           (((1,), (0,)), ((), ())),
                           preferred_element_type=jnp.float32)


def _mlp2_body(x_ref, w1, b1, w2, b2, o_ref):
    h = _sp2(_dot(x_ref[...], w1[...]) + b1[...])
    o_ref[...] = _sp2(_dot(h, w2[...]) + b2[...])


def _mlp2(x, w1, b1, w2, b2, tile):
    n, d = x.shape
    dh, do = w1.shape[1], w2.shape[1]
    return pl.pallas_call(
        _mlp2_body,
        grid=(n // tile,),
        in_specs=[
            pl.BlockSpec((tile, d), lambda i: (i, 0)),
            pl.BlockSpec((d, dh), lambda i: (0, 0)),
            pl.BlockSpec((1, dh), lambda i: (0, 0)),
            pl.BlockSpec((dh, do), lambda i: (0, 0)),
            pl.BlockSpec((1, do), lambda i: (0, 0)),
        ],
        out_specs=pl.BlockSpec((tile, do), lambda i: (i, 0)),
        out_shape=jax.ShapeDtypeStruct((n, do), jnp.float32),
    )(x, w1, b1, w2, b2)


def _node_enc_body(nf_ref, emb, w1, b1, w2, b2, o_ref):
    nf = nf_ref[0, 0, :]
    oh = (lax.broadcasted_iota(jnp.int32, (T_N, 96), 1) == nf[:, None])
    h0 = _dot(oh.astype(jnp.float32), emb[...])
    h = _sp2(_dot(h0, w1[...]) + b1[...])
    o_ref[...] = _sp2(_dot(h, w2[...]) + b2[...])


def _node_enc(node_feat, emb_pad, w1, b1, w2, b2):
    nf3 = node_feat.reshape(NN // T_N, 1, T_N)
    return pl.pallas_call(
        _node_enc_body,
        grid=(NN // T_N,),
        in_specs=[
            pl.BlockSpec((1, 1, T_N), lambda i: (i, 0, 0)),
            pl.BlockSpec((96, 16), lambda i: (0, 0)),
            pl.BlockSpec((16, 64), lambda i: (0, 0)),
            pl.BlockSpec((1, 64), lambda i: (0, 0)),
            pl.BlockSpec((64, 32), lambda i: (0, 0)),
            pl.BlockSpec((1, 32), lambda i: (0, 0)),
        ],
        out_specs=pl.BlockSpec((T_N, 32), lambda i: (i, 0)),
        out_shape=jax.ShapeDtypeStruct((NN, 32), jnp.float32),
    )(nf3, emb_pad, w1, b1, w2, b2)


def _econv_body(vs_ref, vd_ref, e_ref, e0_ref, u_ref,
                wa, wb, wc, wd, b1, w2, b2, w3, b3,
                enew_ref, eres_ref, acc_ref):
    i = pl.program_id(0)
    h = (_dot(vs_ref[...], wa[...]) + _dot(vd_ref[...], wb[...])
         + _dot(e_ref[...], wc[...]) + _dot(u_ref[...], wd[...]) + b1[...])
    h = _sp2(h)
    h = _sp2(_dot(h, w2[...]) + b2[...])
    en = _sp2(_dot(h, w3[...]) + b3[...])
    enew_ref[...] = en
    eres_ref[...] = en + e0_ref[...]
    part = jnp.sum(en, axis=0, keepdims=True)

    @pl.when(i == 0)
    def _():
        acc_ref[...] = part

    @pl.when(i > 0)
    def _():
        acc_ref[...] += part


def _edge_conv(vpair, e_in, e0, u, p):
    (w1, b1), (w2, b2), (w3, b3) = p
    wa, wb, wc, wd = w1[0:32], w1[32:64], w1[64:96], w1[96:128]
    full = lambda s: pl.BlockSpec(s, lambda i: (0, 0))
    return pl.pallas_call(
        _econv_body,
        grid=(NE // T_E,),
        in_specs=[
            pl.BlockSpec((T_E, 32), lambda i: (i, 0)),
            pl.BlockSpec((T_E, 32), lambda i: (i + NE // T_E, 0)),
            pl.BlockSpec((T_E, 32), lambda i: (i, 0)),
            pl.BlockSpec((T_E, 32), lambda i: (i, 0)),
            full((1, 32)),
            full((32, 64)), full((32, 64)), full((32, 64)), full((32, 64)),
            full((1, 64)), full((64, 64)), full((1, 64)),
            full((64, 32)), full((1, 32)),
        ],
        out_specs=[
            pl.BlockSpec((T_E, 32), lambda i: (i, 0)),
            pl.BlockSpec((T_E, 32), lambda i: (i, 0)),
            pl.BlockSpec((1, 32), lambda i: (0, 0)),
        ],
        out_shape=[
            jax.ShapeDtypeStruct((NE, 32), jnp.float32),
            jax.ShapeDtypeStruct((NE, 32), jnp.float32),
            jax.ShapeDtypeStruct((1, 32), jnp.float32),
        ],
    )(vpair, vpair, e_in, e0, u.reshape(1, 32),
      wa, wb, wc, wd, b1.reshape(1, 64), w2, b2.reshape(1, 64),
      w3, b3.reshape(1, 32))


def _nconv_body(v_ref, v0_ref, es0_ref, es1_ref, c0_ref, c1_ref, u_ref,
                wa, wb, wc, b1, w2, b2, w3, b3, vres_ref, acc_ref):
    i = pl.program_id(0)
    cnt = jnp.maximum(c0_ref[0] + c1_ref[0], 1.0)[:, 0:1]
    ve = (es0_ref[0] + es1_ref[0]) / cnt
    h = (_dot(v_ref[...], wa[...]) + _dot(ve, wb[...])
         + _dot(u_ref[...], wc[...]) + b1[...])
    h = _sp2(h)
    h = _sp2(_dot(h, w2[...]) + b2[...])
    vn = _sp2(_dot(h, w3[...]) + b3[...])
    vres_ref[...] = vn + v0_ref[...]
    part = jnp.sum(vn, axis=0, keepdims=True)

    @pl.when(i == 0)
    def _():
        acc_ref[...] = part

    @pl.when(i > 0)
    def _():
        acc_ref[...] += part


def _node_conv(v_in, v0, esum, cnts, u, p):
    (w1, b1), (w2, b2), (w3, b3) = p
    wa, wb, wc = w1[0:32], w1[32:64], w1[64:96]
    full = lambda s: pl.BlockSpec(s, lambda i: (0, 0))
    return pl.pallas_call(
        _nconv_body,
        grid=(NN // T_N,),
        in_specs=[
            pl.BlockSpec((T_N, 32), lambda i: (i, 0)),
            pl.BlockSpec((T_N, 32), lambda i: (i, 0)),
            pl.BlockSpec((1, T_N, 32), lambda i: (0, i, 0)),
            pl.BlockSpec((1, T_N, 32), lambda i: (1, i, 0)),
            pl.BlockSpec((1, T_N, 16), lambda i: (0, i, 0)),
            pl.BlockSpec((1, T_N, 16), lambda i: (1, i, 0)),
            full((1, 32)),
            full((32, 64)), full((32, 64)), full((32, 64)),
            full((1, 64)), full((64, 64)), full((1, 64)),
            full((64, 32)), full((1, 32)),
        ],
        out_specs=[
            pl.BlockSpec((T_N, 32), lambda i: (i, 0)),
            pl.BlockSpec((1, 32), lambda i: (0, 0)),
        ],
        out_shape=[
            jax.ShapeDtypeStruct((NN, 32), jnp.float32),
            jax.ShapeDtypeStruct((1, 32), jnp.float32),
        ],
    )(v_in, v0, esum, esum, cnts, cnts, u.reshape(1, 32),
      wa, wb, wc, b1.reshape(1, 64), w2, b2.reshape(1, 64),
      w3, b3.reshape(1, 32))


def _s2s_body(f_ref, q_ref, o_ref, m_sc, z_sc, r_sc):
    i = pl.program_id(0)

    @pl.when(i == 0)
    def _():
        m_sc[...] = jnp.full((1, 32), -3.0e38, jnp.float32)
        z_sc[...] = jnp.zeros((1, 32), jnp.float32)
        r_sc[...] = jnp.zeros((1, 32), jnp.float32)

    f = f_ref[...]
    s = _dot(f, q_ref[...].reshape(32, 1))[:, 0]
    m_old = jnp.max(m_sc[...])
    m_new = jnp.maximum(m_old, jnp.max(s))
    scale = jnp.exp(jnp.full((1, 32), m_old - m_new, jnp.float32))
    es = jnp.exp(s - m_new)
    r_sc[...] = r_sc[...] * scale + jnp.sum(f * es[:, None], axis=0,
                                            keepdims=True)
    z_sc[...] = z_sc[...] * scale + jnp.sum(es)
    m_sc[...] = jnp.full((1, 32), m_new, jnp.float32)
    o_ref[...] = r_sc[...] / z_sc[...]


def _s2s_pass(feat, q, tile):
    n = feat.shape[0]
    return pl.pallas_call(
        _s2s_body,
        grid=(n // tile,),
        in_specs=[
            pl.BlockSpec((tile, 32), lambda i: (i, 0)),
            pl.BlockSpec((1, 32), lambda i: (0, 0)),
        ],
        out_specs=pl.BlockSpec((1, 32), lambda i: (0, 0)),
        out_shape=jax.ShapeDtypeStruct((1, 32), jnp.float32),
        scratch_shapes=[pltpu.VMEM((1, 32), jnp.float32)] * 3,
    )(feat, q)


def _lstm_cell(x, h, c, p):
    wih, whh, bih, bhh = p
    g = x @ wih.T + bih + h @ whh.T + bhh
    i, f, gg, o = jnp.split(g, 4, axis=-1)
    c = jax.nn.sigmoid(f) * c + jax.nn.sigmoid(i) * jnp.tanh(gg)
    h = jax.nn.sigmoid(o) * jnp.tanh(c)
    return h, c


def _set2set(feat, p, tile):
    h = jnp.zeros((1, 32), jnp.float32)
    c = jnp.zeros((1, 32), jnp.float32)
    q_star = jnp.zeros((1, 64), jnp.float32)
    for _ in range(2):
        h, c = _lstm_cell(q_star, h, c, p)
        readout = _s2s_pass(feat, h, tile)
        q_star = jnp.concatenate([h, readout], axis=-1)
    return q_star


_MESH = dict(core_axis_name="c", subcore_axis_name="s")
_SC_PARAMS = pltpu.CompilerParams(use_tc_tiling_on_sc=False)


def _sc_gather(table, idx2d):
    nchp = idx2d.shape[0]
    ni = nchp * 128
    cpw = nchp // 32
    nsup = cpw // 8
    mesh = plsc.VectorSubcoreMesh(**_MESH)

    @functools.partial(
        pl.kernel, mesh=mesh, compiler_params=_SC_PARAMS,
        out_type=jax.ShapeDtypeStruct((ni, 32), jnp.float32),
        scratch_types=[
            pltpu.VMEM((2, 8, 128), jnp.int32),
            pltpu.VMEM((2, 1024, 32), jnp.float32),
            pltpu.SemaphoreType.DMA,
            pltpu.SemaphoreType.DMA,
            pltpu.SemaphoreType.DMA,
        ],
    )
    def k(table_hbm, idx_hbm, out_hbm, idx_v, rows_v, semi, semg, semo):
        w = lax.axis_index("s") * 2 + lax.axis_index("c")
        c0 = w * cpw

        pltpu.async_copy(idx_hbm.at[pl.ds(c0, 8)], idx_v.at[0], semi)
        pltpu.async_copy(idx_hbm.at[pl.ds(c0 + 8, 8)], idx_v.at[1], semi)

        @pl.loop(0, nsup + 1, step=2)
        def _(t):
            for b in range(2):
                sup = t + b

                @pl.when(sup < nsup)
                def _():
                    pltpu.make_async_copy(idx_hbm.at[pl.ds(c0 + sup * 8, 8)],
                                          idx_v.at[b], semi).wait()
                    @pl.when(sup >= 2)
                    def _():
                        pltpu.make_async_copy(
                            rows_v.at[b],
                            out_hbm.at[pl.ds((c0 + (sup - 2) * 8) * 128,
                                             1024)],
                            semo).wait()

                    for j in range(8):
                        pltpu.async_copy(
                            table_hbm.at[idx_v.at[b, j]],
                            rows_v.at[b, pl.ds(j * 128, 128)], semg)
                    pltpu.make_async_copy(out_hbm.at[pl.ds(0, 1024)],
                                          rows_v.at[b], semg).wait()

                    @pl.when(sup + 2 < nsup)
                    def _():
                        pltpu.async_copy(
                            idx_hbm.at[pl.ds(c0 + (sup + 2) * 8, 8)],
                            idx_v.at[b], semi)

                    pltpu.async_copy(
                        rows_v.at[b],
                        out_hbm.at[pl.ds((c0 + sup * 8) * 128, 1024)], semo)

        for sup in (nsup - 2, nsup - 1):
            pltpu.make_async_copy(
                rows_v.at[sup % 2],
                out_hbm.at[pl.ds((c0 + sup * 8) * 128, 1024)], semo).wait()

    return k(table, idx2d)


def _sc_scatter(values, idx, zeros400):
    nch = NE // 128
    tmax = (nch + 31) // 32
    mesh = plsc.VectorSubcoreMesh(**_MESH)

    @functools.partial(
        pl.kernel, mesh=mesh, compiler_params=_SC_PARAMS,
        out_type=jax.ShapeDtypeStruct((2, NN, 32), jnp.float32),
        scratch_types=[
            pltpu.VMEM((128,), jnp.int32),
            pltpu.VMEM((128, 32), jnp.float32),
            pltpu.VMEM_SHARED((NN, 32), jnp.float32),
            pltpu.SemaphoreType.DMA,
        ],
    )
    def k(val_hbm, idx_hbm, z_hbm, out_hbm, idx_v, rows_v, acc_sh, sem):
        cid = lax.axis_index("c")
        sid = lax.axis_index("s")
        w = sid * 2 + cid

        @pl.loop(0, 8)
        def _(t):
            cz = sid + t * 16

            @pl.when(cz < 125)
            def _():
                pltpu.sync_copy(z_hbm, acc_sh.at[pl.ds(cz * 400, 400)])

        plsc.subcore_barrier()

        @pl.loop(0, tmax)
        def _(t):
            ch = w + t * 32

            @pl.when(ch < nch)
            def _():
                base = ch * 128
                pltpu.sync_copy(idx_hbm.at[pl.ds(base, 128)], idx_v)
                pltpu.sync_copy(val_hbm.at[pl.ds(base, 128)], rows_v)
                pltpu.sync_copy(rows_v, acc_sh.at[idx_v], add=True)

        plsc.subcore_barrier()

        @pl.loop(0, 8)
        def _(t):
            co = sid + t * 16

            @pl.when(co < 125)
            def _():
                pltpu.sync_copy(acc_sh.at[pl.ds(co * 400, 400)],
                                out_hbm.at[cid, pl.ds(co * 400, 400)])

    return k(values, idx, zeros400)


def _sc_count(idx, ones128, zeros400):
    nch = NE // 128
    tmax = (nch + 31) // 32
    mesh = plsc.VectorSubcoreMesh(**_MESH)

    @functools.partial(
        pl.kernel, mesh=mesh, compiler_params=_SC_PARAMS,
        out_type=jax.ShapeDtypeStruct((2, NN, 16), jnp.float32),
        scratch_types=[
            pltpu.VMEM((128,), jnp.int32),
            pltpu.VMEM((128, 16), jnp.float32),
            pltpu.VMEM_SHARED((NN, 16), jnp.float32),
            pltpu.SemaphoreType.DMA,
        ],
    )
    def k(idx_hbm, ones_hbm, z_hbm, out_hbm, idx_v, ones_v, acc_sh, sem):
        cid = lax.axis_index("c")
        sid = lax.axis_index("s")
        w = sid * 2 + cid
        pltpu.sync_copy(ones_hbm, ones_v)

        @pl.loop(0, 8)
        def _(t):
            cz = sid + t * 16

            @pl.when(cz < 125)
            def _():
                pltpu.sync_copy(z_hbm, acc_sh.at[pl.ds(cz * 400, 400)])

        plsc.subcore_barrier()

        @pl.loop(0, tmax)
        def _(t):
            ch = w + t * 32

            @pl.when(ch < nch)
            def _():
                base = ch * 128
                pltpu.sync_copy(idx_hbm.at[pl.ds(base, 128)], idx_v)
                pltpu.sync_copy(ones_v, acc_sh.at[idx_v], add=True)

        plsc.subcore_barrier()

        @pl.loop(0, 8)
        def _(t):
            co = sid + t * 16

            @pl.when(co < 125)
            def _():
                pltpu.sync_copy(acc_sh.at[pl.ds(co * 400, 400)],
                                out_hbm.at[cid, pl.ds(co * 400, 400)])

    return k(idx, ones128, zeros400)


def _mlp_jnp(x, layers, activate_last=True):
    n = len(layers)
    for i, (w, b) in enumerate(layers):
        x = x @ w + b
        if i < n - 1 or activate_last:
            x = _sp2(x)
    return x


def kernel(edge_index, node_feat, edge_feat, state_feat, params):
    p = params
    idx_all = edge_index.reshape(2 * NE).astype(jnp.int32)
    nchp = 32 * 392
    idx_pad = jnp.zeros((nchp * 128,), jnp.int32).at[0:2 * NE].set(idx_all)
    idx2d = idx_pad.reshape(nchp, 128)
    dst = edge_index[1].astype(jnp.int32)

    zeros400_32 = jnp.zeros((400, 32), jnp.float32)
    zeros400_16 = jnp.zeros((400, 16), jnp.float32)
    ones128_16 = jnp.ones((128, 16), jnp.float32)

    (we1, be1), (we2, be2) = p['edge_enc']
    e = _mlp2(edge_feat, we1, be1.reshape(1, -1), we2, be2.reshape(1, -1), T_E)
    emb_pad = jnp.zeros((96, 16), jnp.float32).at[0:89].set(p['node_emb'])
    (wn1, bn1), (wn2, bn2) = p['node_enc']
    v = _node_enc(node_feat.astype(jnp.int32), emb_pad,
                  wn1, bn1.reshape(1, -1), wn2, bn2.reshape(1, -1))
    u = _mlp_jnp(state_feat, p['state_enc'])

    cnts = _sc_count(dst, ones128_16, zeros400_16)

    for blk in p['blocks']:
        e0, v0, u0 = e, v, u
        if blk['dense_e']:
            (wd1, bd1), (wd2, bd2) = blk['dense_e']
            e_d = _mlp2(e, wd1, bd1.reshape(1, -1), wd2, bd2.reshape(1, -1),
                        T_E)
            (wv1, bv1), (wv2, bv2) = blk['dense_n']
            v_d = _mlp2(v, wv1, bv1.reshape(1, -1), wv2, bv2.reshape(1, -1),
                        T_N)
            u_d = _mlp_jnp(u, blk['dense_s'])
        else:
            e_d, v_d, u_d = e, v, u

        vpair = _sc_gather(v_d, idx2d)
        e_new, e_res, e_acc = _edge_conv(vpair, e_d, e0, u_d, blk['conv_e'])
        esum = _sc_scatter(e_new, dst, zeros400_32)
        v_new, v_acc = _node_conv(v_d, v0, esum, cnts, u_d, blk['conv_n'])

        e_mean = e_acc / float(NE)
        v_mean = v_acc / float(NN)
        u_new = _mlp_jnp(jnp.concatenate([u_d, e_mean, v_mean], axis=-1),
                         blk['conv_s'])
        e, v, u = e_res, v_new, u_new + u0

    node_vec = _set2set(v, p['lstm_node'], T_N)
    edge_vec = _set2set(e, p['lstm_edge'], T_E)
    vec = jnp.concatenate([node_vec[0], edge_vec[0], u[0]], axis=-1)
    out = _mlp_jnp(vec[None, :], p['out_proj'], activate_last=False)
    return out[0]


# speedup vs baseline: 1.1440x; 1.0719x over previous
import functools

import jax
import jax.numpy as jnp
import numpy as np
from jax import lax
from jax.experimental import pallas as pl
from jax.experimental.pallas import tpu as pltpu
from jax.experimental.pallas import tpu_sc as plsc

LOG2 = float(np.log(2.0))
NN = 50000
NE = 800000
T_E = 4000
T_N = 2000


def _sp2(x):
    return jnp.logaddexp(x, 0.0) - LOG2


def _sp2h(x):
    xb = x.astype(jnp.bfloat16)
    return jnp.logaddexp(xb, jnp.bfloat16(0.0)) - jnp.bfloat16(LOG2)


def _bf16x3(a, b):
    ah = a.astype(jnp.bfloat16)
    al = (a - ah.astype(jnp.float32)).astype(jnp.bfloat16)
    bh = b.astype(jnp.bfloat16)
    bl = (b - bh.astype(jnp.float32)).astype(jnp.bfloat16)
    d = lambda x, y: lax.dot_general(x, y, (((1,), (0,)), ((), ())),
                                     preferred_element_type=jnp.float32)
    return d(ah, bh) + d(ah, bl) + d(al, bh)


def _dot(a, b):
    return lax.dot_general(a.astype(jnp.bfloat16), b.astype(jnp.bfloat16),
                           (((1,), (0,)), ((), ())),
                           preferred_element_type=jnp.float32)


def _mlp2_body(x_ref, w1, b1, w2, b2, o_ref):
    h = _sp2h(_dot(x_ref[...], w1[...]) + b1[...])
    o_ref[...] = _sp2(_dot(h, w2[...]) + b2[...])


def _mlp2(x, w1, b1, w2, b2, tile):
    n, d = x.shape
    dh, do = w1.shape[1], w2.shape[1]
    return pl.pallas_call(
        _mlp2_body,
        grid=(n // tile,),
        in_specs=[
            pl.BlockSpec((tile, d), lambda i: (i, 0)),
            pl.BlockSpec((d, dh), lambda i: (0, 0)),
            pl.BlockSpec((1, dh), lambda i: (0, 0)),
            pl.BlockSpec((dh, do), lambda i: (0, 0)),
            pl.BlockSpec((1, do), lambda i: (0, 0)),
        ],
        out_specs=pl.BlockSpec((tile, do), lambda i: (i, 0)),
        out_shape=jax.ShapeDtypeStruct((n, do), jnp.float32),
    )(x, w1, b1, w2, b2)


def _node_enc_body(nf_ref, emb, w1, b1, w2, b2, o_ref):
    nf = nf_ref[0, 0, :]
    oh = (lax.broadcasted_iota(jnp.int32, (T_N, 96), 1) == nf[:, None])
    h0 = _dot(oh.astype(jnp.float32), emb[...])
    h = _sp2h(_dot(h0, w1[...]) + b1[...])
    o_ref[...] = _sp2(_dot(h, w2[...]) + b2[...])


def _node_enc(node_feat, emb_pad, w1, b1, w2, b2):
    nf3 = node_feat.reshape(NN // T_N, 1, T_N)
    return pl.pallas_call(
        _node_enc_body,
        grid=(NN // T_N,),
        in_specs=[
            pl.BlockSpec((1, 1, T_N), lambda i: (i, 0, 0)),
            pl.BlockSpec((96, 16), lambda i: (0, 0)),
            pl.BlockSpec((16, 64), lambda i: (0, 0)),
            pl.BlockSpec((1, 64), lambda i: (0, 0)),
            pl.BlockSpec((64, 32), lambda i: (0, 0)),
            pl.BlockSpec((1, 32), lambda i: (0, 0)),
        ],
        out_specs=pl.BlockSpec((T_N, 32), lambda i: (i, 0)),
        out_shape=jax.ShapeDtypeStruct((NN, 32), jnp.float32),
    )(nf3, emb_pad, w1, b1, w2, b2)


def _econv_body(vs_ref, vd_ref, e_ref, e0_ref, u_ref,
                wa, wb, wc, wd, b1, w2, b2, w3, b3,
                enew_ref, eres_ref, acc_ref):
    i = pl.program_id(0)
    h = (_dot(vs_ref[...], wa[...]) + _dot(vd_ref[...], wb[...])
         + _dot(e_ref[...], wc[...]) + _dot(u_ref[...], wd[...]) + b1[...])
    h = _sp2h(h)
    h = _sp2h(_dot(h, w2[...]) + b2[...])
    en = _sp2(_dot(h, w3[...]) + b3[...])
    enew_ref[...] = en
    eres_ref[...] = en + e0_ref[...]
    part = jnp.sum(en, axis=0, keepdims=True)

    @pl.when(i == 0)
    def _():
        acc_ref[...] = part

    @pl.when(i > 0)
    def _():
        acc_ref[...] += part


def _edge_conv(vpair, e_in, e0, u, p):
    (w1, b1), (w2, b2), (w3, b3) = p
    wa, wb, wc, wd = w1[0:32], w1[32:64], w1[64:96], w1[96:128]
    full = lambda s: pl.BlockSpec(s, lambda i: (0, 0))
    return pl.pallas_call(
        _econv_body,
        grid=(NE // T_E,),
        in_specs=[
            pl.BlockSpec((T_E, 32), lambda i: (i, 0)),
            pl.BlockSpec((T_E, 32), lambda i: (i + NE // T_E, 0)),
            pl.BlockSpec((T_E, 32), lambda i: (i, 0)),
            pl.BlockSpec((T_E, 32), lambda i: (i, 0)),
            full((1, 32)),
            full((32, 64)), full((32, 64)), full((32, 64)), full((32, 64)),
            full((1, 64)), full((64, 64)), full((1, 64)),
            full((64, 32)), full((1, 32)),
        ],
        out_specs=[
            pl.BlockSpec((T_E, 32), lambda i: (i, 0)),
            pl.BlockSpec((T_E, 32), lambda i: (i, 0)),
            pl.BlockSpec((1, 32), lambda i: (0, 0)),
        ],
        out_shape=[
            jax.ShapeDtypeStruct((NE, 32), jnp.float32),
            jax.ShapeDtypeStruct((NE, 32), jnp.float32),
            jax.ShapeDtypeStruct((1, 32), jnp.float32),
        ],
    )(vpair, vpair, e_in, e0, u.reshape(1, 32),
      wa, wb, wc, wd, b1.reshape(1, 64), w2, b2.reshape(1, 64),
      w3, b3.reshape(1, 32))


def _nconv_body(v_ref, v0_ref, es0_ref, es1_ref, c0_ref, c1_ref, u_ref,
                wa, wb, wc, b1, w2, b2, w3, b3, vres_ref, acc_ref):
    i = pl.program_id(0)
    cnt = jnp.maximum(c0_ref[0] + c1_ref[0], 1.0)[:, 0:1]
    ve = (es0_ref[0] + es1_ref[0]) / cnt
    h = (_dot(v_ref[...], wa[...]) + _dot(ve, wb[...])
         + _dot(u_ref[...], wc[...]) + b1[...])
    h = _sp2h(h)
    h = _sp2h(_dot(h, w2[...]) + b2[...])
    vn = _sp2(_dot(h, w3[...]) + b3[...])
    vres_ref[...] = vn + v0_ref[...]
    part = jnp.sum(vn, axis=0, keepdims=True)

    @pl.when(i == 0)
    def _():
        acc_ref[...] = part

    @pl.when(i > 0)
    def _():
        acc_ref[...] += part


def _node_conv(v_in, v0, esum, cnts, u, p):
    (w1, b1), (w2, b2), (w3, b3) = p
    wa, wb, wc = w1[0:32], w1[32:64], w1[64:96]
    full = lambda s: pl.BlockSpec(s, lambda i: (0, 0))
    return pl.pallas_call(
        _nconv_body,
        grid=(NN // T_N,),
        in_specs=[
            pl.BlockSpec((T_N, 32), lambda i: (i, 0)),
            pl.BlockSpec((T_N, 32), lambda i: (i, 0)),
            pl.BlockSpec((1, T_N, 32), lambda i: (0, i, 0)),
            pl.BlockSpec((1, T_N, 32), lambda i: (1, i, 0)),
            pl.BlockSpec((1, T_N, 16), lambda i: (0, i, 0)),
            pl.BlockSpec((1, T_N, 16), lambda i: (1, i, 0)),
            full((1, 32)),
            full((32, 64)), full((32, 64)), full((32, 64)),
            full((1, 64)), full((64, 64)), full((1, 64)),
            full((64, 32)), full((1, 32)),
        ],
        out_specs=[
            pl.BlockSpec((T_N, 32), lambda i: (i, 0)),
            pl.BlockSpec((1, 32), lambda i: (0, 0)),
        ],
        out_shape=[
            jax.ShapeDtypeStruct((NN, 32), jnp.float32),
            jax.ShapeDtypeStruct((1, 32), jnp.float32),
        ],
    )(v_in, v0, esum, esum, cnts, cnts, u.reshape(1, 32),
      wa, wb, wc, b1.reshape(1, 64), w2, b2.reshape(1, 64),
      w3, b3.reshape(1, 32))


def _s2s_body(f_ref, q_ref, o_ref, m_sc, z_sc, r_sc):
    i = pl.program_id(0)

    @pl.when(i == 0)
    def _():
        m_sc[...] = jnp.full((1, 32), -3.0e38, jnp.float32)
        z_sc[...] = jnp.zeros((1, 32), jnp.float32)
        r_sc[...] = jnp.zeros((1, 32), jnp.float32)

    f = f_ref[...]
    s = _dot(f, q_ref[...].reshape(32, 1))[:, 0]
    m_old = jnp.max(m_sc[...])
    m_new = jnp.maximum(m_old, jnp.max(s))
    scale = jnp.exp(jnp.full((1, 32), m_old - m_new, jnp.float32))
    es = jnp.exp(s - m_new)
    r_sc[...] = r_sc[...] * scale + jnp.sum(f * es[:, None], axis=0,
                                            keepdims=True)
    z_sc[...] = z_sc[...] * scale + jnp.sum(es)
    m_sc[...] = jnp.full((1, 32), m_new, jnp.float32)
    o_ref[...] = r_sc[...] / z_sc[...]


def _s2s_pass(feat, q, tile):
    n = feat.shape[0]
    return pl.pallas_call(
        _s2s_body,
        grid=(n // tile,),
        in_specs=[
            pl.BlockSpec((tile, 32), lambda i: (i, 0)),
            pl.BlockSpec((1, 32), lambda i: (0, 0)),
        ],
        out_specs=pl.BlockSpec((1, 32), lambda i: (0, 0)),
        out_shape=jax.ShapeDtypeStruct((1, 32), jnp.float32),
        scratch_shapes=[pltpu.VMEM((1, 32), jnp.float32)] * 3,
    )(feat, q)


def _lstm_cell(x, h, c, p):
    wih, whh, bih, bhh = p
    g = x @ wih.T + bih + h @ whh.T + bhh
    i, f, gg, o = jnp.split(g, 4, axis=-1)
    c = jax.nn.sigmoid(f) * c + jax.nn.sigmoid(i) * jnp.tanh(gg)
    h = jax.nn.sigmoid(o) * jnp.tanh(c)
    return h, c


def _set2set(feat, p, tile):
    h = jnp.zeros((1, 32), jnp.float32)
    c = jnp.zeros((1, 32), jnp.float32)
    q_star = jnp.zeros((1, 64), jnp.float32)
    for _ in range(2):
        h, c = _lstm_cell(q_star, h, c, p)
        readout = _s2s_pass(feat, h, tile)
        q_star = jnp.concatenate([h, readout], axis=-1)
    return q_star


_MESH = dict(core_axis_name="c", subcore_axis_name="s")
_SC_PARAMS = pltpu.CompilerParams(use_tc_tiling_on_sc=False)


def _sc_gather(table, idx2d):
    nchp = idx2d.shape[0]
    ni = nchp * 128
    cpw = nchp // 32
    nsup = cpw // 8
    mesh = plsc.VectorSubcoreMesh(**_MESH)

    @functools.partial(
        pl.kernel, mesh=mesh, compiler_params=_SC_PARAMS,
        out_type=jax.ShapeDtypeStruct((ni, 32), jnp.float32),
        scratch_types=[
            pltpu.VMEM((2, 8, 128), jnp.int32),
            pltpu.VMEM((2, 1024, 32), jnp.float32),
            pltpu.SemaphoreType.DMA,
            pltpu.SemaphoreType.DMA,
            pltpu.SemaphoreType.DMA,
        ],
    )
    def k(table_hbm, idx_hbm, out_hbm, idx_v, rows_v, semi, semg, semo):
        w = lax.axis_index("s") * 2 + lax.axis_index("c")
        c0 = w * cpw

        pltpu.async_copy(idx_hbm.at[pl.ds(c0, 8)], idx_v.at[0], semi)
        pltpu.async_copy(idx_hbm.at[pl.ds(c0 + 8, 8)], idx_v.at[1], semi)

        @pl.loop(0, nsup + 1, step=2)
        def _(t):
            for b in range(2):
                sup = t + b

                @pl.when(sup < nsup)
                def _():
                    pltpu.make_async_copy(idx_hbm.at[pl.ds(c0 + sup * 8, 8)],
                                          idx_v.at[b], semi).wait()
                    @pl.when(sup >= 2)
                    def _():
                        pltpu.make_async_copy(
                            rows_v.at[b],
                            out_hbm.at[pl.ds((c0 + (sup - 2) * 8) * 128,
                                             1024)],
                            semo).wait()

                    for j in range(8):
                        pltpu.async_copy(
                            table_hbm.at[idx_v.at[b, j]],
                            rows_v.at[b, pl.ds(j * 128, 128)], semg)
                    pltpu.make_async_copy(out_hbm.at[pl.ds(0, 1024)],
                                          rows_v.at[b], semg).wait()

                    @pl.when(sup + 2 < nsup)
                    def _():
                        pltpu.async_copy(
                            idx_hbm.at[pl.ds(c0 + (sup + 2) * 8, 8)],
                            idx_v.at[b], semi)

                    pltpu.async_copy(
                        rows_v.at[b],
                        out_hbm.at[pl.ds((c0 + sup * 8) * 128, 1024)], semo)

        for sup in (nsup - 2, nsup - 1):
            pltpu.make_async_copy(
                rows_v.at[sup % 2],
                out_hbm.at[pl.ds((c0 + sup * 8) * 128, 1024)], semo).wait()

    return k(table, idx2d)


def _sc_scatter(values, idx, zeros400):
    nch = NE // 128
    tmax = (nch + 31) // 32
    mesh = plsc.VectorSubcoreMesh(**_MESH)

    @functools.partial(
        pl.kernel, mesh=mesh, compiler_params=_SC_PARAMS,
        out_type=jax.ShapeDtypeStruct((2, NN, 32), jnp.float32),
        scratch_types=[
            pltpu.VMEM((128,), jnp.int32),
            pltpu.VMEM((128, 32), jnp.float32),
            pltpu.VMEM_SHARED((NN, 32), jnp.float32),
            pltpu.SemaphoreType.DMA,
        ],
    )
    def k(val_hbm, idx_hbm, z_hbm, out_hbm, idx_v, rows_v, acc_sh, sem):
        cid = lax.axis_index("c")
        sid = lax.axis_index("s")
        w = sid * 2 + cid

        @pl.loop(0, 8)
        def _(t):
            cz = sid + t * 16

            @pl.when(cz < 125)
            def _():
                pltpu.sync_copy(z_hbm, acc_sh.at[pl.ds(cz * 400, 400)])

        plsc.subcore_barrier()

        @pl.loop(0, tmax)
        def _(t):
            ch = w + t * 32

            @pl.when(ch < nch)
            def _():
                base = ch * 128
                pltpu.sync_copy(idx_hbm.at[pl.ds(base, 128)], idx_v)
                pltpu.sync_copy(val_hbm.at[pl.ds(base, 128)], rows_v)
                pltpu.sync_copy(rows_v, acc_sh.at[idx_v], add=True)

        plsc.subcore_barrier()

        @pl.loop(0, 8)
        def _(t):
            co = sid + t * 16

            @pl.when(co < 125)
            def _():
                pltpu.sync_copy(acc_sh.at[pl.ds(co * 400, 400)],
                                out_hbm.at[cid, pl.ds(co * 400, 400)])

    return k(values, idx, zeros400)


def _sc_count(idx, ones128, zeros400):
    nch = NE // 128
    tmax = (nch + 31) // 32
    mesh = plsc.VectorSubcoreMesh(**_MESH)

    @functools.partial(
        pl.kernel, mesh=mesh, compiler_params=_SC_PARAMS,
        out_type=jax.ShapeDtypeStruct((2, NN, 16), jnp.float32),
        scratch_types=[
            pltpu.VMEM((128,), jnp.int32),
            pltpu.VMEM((128, 16), jnp.float32),
            pltpu.VMEM_SHARED((NN, 16), jnp.float32),
            pltpu.SemaphoreType.DMA,
        ],
    )
    def k(idx_hbm, ones_hbm, z_hbm, out_hbm, idx_v, ones_v, acc_sh, sem):
        cid = lax.axis_index("c")
        sid = lax.axis_index("s")
        w = sid * 2 + cid
        pltpu.sync_copy(ones_hbm, ones_v)

        @pl.loop(0, 8)
        def _(t):
            cz = sid + t * 16

            @pl.when(cz < 125)
            def _():
                pltpu.sync_copy(z_hbm, acc_sh.at[pl.ds(cz * 400, 400)])

        plsc.subcore_barrier()

        @pl.loop(0, tmax)
        def _(t):
            ch = w + t * 32

            @pl.when(ch < nch)
            def _():
                base = ch * 128
                pltpu.sync_copy(idx_hbm.at[pl.ds(base, 128)], idx_v)
                pltpu.sync_copy(ones_v, acc_sh.at[idx_v], add=True)

        plsc.subcore_barrier()

        @pl.loop(0, 8)
        def _(t):
            co = sid + t * 16

            @pl.when(co < 125)
            def _():
                pltpu.sync_copy(acc_sh.at[pl.ds(co * 400, 400)],
                                out_hbm.at[cid, pl.ds(co * 400, 400)])

    return k(idx, ones128, zeros400)


def _mlp_jnp(x, layers, activate_last=True):
    n = len(layers)
    for i, (w, b) in enumerate(layers):
        x = x @ w + b
        if i < n - 1 or activate_last:
            x = _sp2(x)
    return x


def kernel(edge_index, node_feat, edge_feat, state_feat, params):
    p = params
    idx_all = edge_index.reshape(2 * NE).astype(jnp.int32)
    nchp = 32 * 392
    idx_pad = jnp.zeros((nchp * 128,), jnp.int32).at[0:2 * NE].set(idx_all)
    idx2d = idx_pad.reshape(nchp, 128)
    dst = edge_index[1].astype(jnp.int32)

    zeros400_32 = jnp.zeros((400, 32), jnp.float32)
    zeros400_16 = jnp.zeros((400, 16), jnp.float32)
    ones128_16 = jnp.ones((128, 16), jnp.float32)

    (we1, be1), (we2, be2) = p['edge_enc']
    e = _mlp2(edge_feat, we1, be1.reshape(1, -1), we2, be2.reshape(1, -1), T_E)
    emb_pad = jnp.zeros((96, 16), jnp.float32).at[0:89].set(p['node_emb'])
    (wn1, bn1), (wn2, bn2) = p['node_enc']
    v = _node_enc(node_feat.astype(jnp.int32), emb_pad,
                  wn1, bn1.reshape(1, -1), wn2, bn2.reshape(1, -1))
    u = _mlp_jnp(state_feat, p['state_enc'])

    cnts = _sc_count(dst, ones128_16, zeros400_16)

    for blk in p['blocks']:
        e0, v0, u0 = e, v, u
        if blk['dense_e']:
            (wd1, bd1), (wd2, bd2) = blk['dense_e']
            e_d = _mlp2(e, wd1, bd1.reshape(1, -1), wd2, bd2.reshape(1, -1),
                        T_E)
            (wv1, bv1), (wv2, bv2) = blk['dense_n']
            v_d = _mlp2(v, wv1, bv1.reshape(1, -1), wv2, bv2.reshape(1, -1),
                        T_N)
            u_d = _mlp_jnp(u, blk['dense_s'])
        else:
            e_d, v_d, u_d = e, v, u

        vpair = _sc_gather(v_d, idx2d)
        e_new, e_res, e_acc = _edge_conv(vpair, e_d, e0, u_d, blk['conv_e'])
        esum = _sc_scatter(e_new, dst, zeros400_32)
        v_new, v_acc = _node_conv(v_d, v0, esum, cnts, u_d, blk['conv_n'])

        e_mean = e_acc / float(NE)
        v_mean = v_acc / float(NN)
        u_new = _mlp_jnp(jnp.concatenate([u_d, e_mean, v_mean], axis=-1),
                         blk['conv_s'])
        e, v, u = e_res, v_new, u_new + u0

    node_vec = _set2set(v, p['lstm_node'], T_N)
    edge_vec = _set2set(e, p['lstm_edge'], T_E)
    vec = jnp.concatenate([node_vec[0], edge_vec[0], u[0]], axis=-1)
    out = _mlp_jnp(vec[None, :], p['out_proj'], activate_last=False)
    return out[0]
